# Initial kernel scaffold; baseline (speedup 1.0000x reference)
import jax
import jax.numpy as jnp
from jax.experimental import pallas as pl


def kernel(feat, edge_index, W, b):
    raise NotImplementedError("write your pallas kernel here")


# speedup vs baseline: 8.4448x; 8.4448x over previous
import functools

import jax
import jax.numpy as jnp
from jax import lax
from jax.experimental import pallas as pl
from jax.experimental.pallas import tpu as pltpu
from jax.experimental.pallas import tpu_sc as plsc

_N = 10000
_E = 320000
_D = 128
_DQ = _D // 4

_NC = 2
_NS = 16
_NW = _NC * _NS
_CH = 80

_EPW = _E // _NW
_NCHD = _EPW // _CH

_EPS = _E // _NS
_NCH = _EPS // _CH

_NPS = _N // _NS

_mesh = plsc.VectorSubcoreMesh(core_axis_name="c", subcore_axis_name="s")


@functools.partial(
    pl.kernel,
    out_type=jax.ShapeDtypeStruct((_NC, _NS, _NPS, 16), jnp.float32),
    mesh=_mesh,
    scratch_types=[
        pltpu.VMEM((_NCHD, _CH), jnp.int32),
        pltpu.VMEM((_CH, 16), jnp.float32),
        pltpu.VMEM_SHARED((_N, 16), jnp.float32),
        pltpu.SemaphoreType.DMA,
    ],
    compiler_params=pltpu.CompilerParams(use_tc_tiling_on_sc=False),
)
def _sc_degree(dst_hbm, ones_hbm, zeros_hbm, out_hbm, didx, ones_v, acc, dsem):
    cid = lax.axis_index("c")
    sid = lax.axis_index("s")
    wid = sid * _NC + cid
    pltpu.sync_copy(ones_hbm, ones_v)
    pltpu.sync_copy(dst_hbm.at[wid], didx)
    pltpu.sync_copy(zeros_hbm, acc.at[pl.ds(sid * _NPS, _NPS)])
    plsc.subcore_barrier()

    _W = 16

    def dstart(c):
        pltpu.async_copy(ones_v, acc.at[didx.at[c]], dsem, add=True)

    def dwait(c):
        pltpu.make_async_copy(ones_v, acc.at[didx.at[c]], dsem).wait()

    for c in range(_W):
        dstart(c)

    def body(c, carry):
        dstart(c)
        dwait(c - _W)
        return carry

    lax.fori_loop(_W, _NCHD, body, 0)
    for c in range(_NCHD - _W, _NCHD):
        dwait(c)
    plsc.subcore_barrier()
    pltpu.sync_copy(acc.at[pl.ds(sid * _NPS, _NPS)], out_hbm.at[cid, sid])


@functools.partial(
    pl.kernel,
    out_type=jax.ShapeDtypeStruct((_NC, 2, _NS, _NPS, _DQ), jnp.float32),
    mesh=_mesh,
    scratch_types=[
        pltpu.VMEM((_EPS,), jnp.int32),
        pltpu.VMEM((_NCH, _CH), jnp.int32),
        pltpu.VMEM((9, _CH, _DQ), jnp.float32),
        pltpu.VMEM_SHARED((_N, _DQ), jnp.float32),
        pltpu.VMEM_SHARED((_N, _DQ), jnp.float32),
        [pltpu.SemaphoreType.DMA] * 9,
        [pltpu.SemaphoreType.DMA] * 9,
    ],
    compiler_params=pltpu.CompilerParams(use_tc_tiling_on_sc=False),
)
def _sc_hop(g4_hbm, src_hbm, dst_hbm, zeros_hbm, out_hbm,
            sidx, didx, rows, gcache, acc, gsems, ssems):
    cid = lax.axis_index("c")
    sid = lax.axis_index("s")
    cp_s = pltpu.async_copy(src_hbm.at[pl.ds(sid * _EPS, _EPS)], sidx, gsems[0])
    cp_d = pltpu.async_copy(dst_hbm.at[sid], didx, gsems[1])
    cp_s.wait()
    cp_d.wait()

    _R = 9
    _G = 7

    def start_gather(c, b):
        pltpu.async_copy(gcache.at[sidx.at[pl.ds(c * _CH, _CH)]],
                         rows.at[b], gsems[b])

    def wait_gather(c, b):
        pltpu.make_async_copy(gcache.at[sidx.at[pl.ds(c * _CH, _CH)]],
                              rows.at[b], gsems[b]).wait()

    def start_scatter(c, b):
        pltpu.async_copy(rows.at[b], acc.at[didx.at[c]], ssems[b], add=True)

    def wait_scatter(c, b):
        pltpu.make_async_copy(rows.at[b], acc.at[didx.at[c]], ssems[b]).wait()

    for q in range(2):
        pltpu.sync_copy(g4_hbm.at[cid, q, pl.ds(sid * _NPS, _NPS)],
                        gcache.at[pl.ds(sid * _NPS, _NPS)])
        pltpu.sync_copy(zeros_hbm, acc.at[pl.ds(sid * _NPS, _NPS)])
        plsc.subcore_barrier()

        for c in range(_G):
            start_gather(c, c % _R)
        for c in range(_G):
            if c >= _R - _G:
                wait_scatter(c - (_R - _G), (c - (_R - _G)) % _R)
            start_gather(c + _G, (c + _G) % _R)
            wait_gather(c, c % _R)
            start_scatter(c, c % _R)

        n_steady = (_NCH - 2 * _G) // _R * _R

        def body(i, carry):
            for j in range(_R):
                c = i * _R + j + _G
                bw = (j + 2 * _G) % _R
                wait_scatter(c - (_R - _G), bw)
                start_gather(c + _G, bw)
                b = (j + _G) % _R
                wait_gather(c, b)
                start_scatter(c, b)
            return carry

        lax.fori_loop(0, n_steady // _R, body, 0)
        for c in range(_G + n_steady, _NCH):
            wait_scatter(c - (_R - _G), (c - (_R - _G)) % _R)
            if c + _G < _NCH:
                start_gather(c + _G, (c + _G) % _R)
            wait_gather(c, c % _R)
            start_scatter(c, c % _R)
        for c in range(_NCH - (_R - _G), _NCH):
            wait_scatter(c, c % _R)
        plsc.subcore_barrier()
        pltpu.sync_copy(acc.at[pl.ds(sid * _NPS, _NPS)], out_hbm.at[cid, q, sid])
        plsc.subcore_barrier()


def _tc_prep(degw, feat):
    def body(degw_ref, feat_ref, norm_ref, g4_ref):
        deg = degw_ref[0, :, 0:1] + degw_ref[1, :, 0:1]
        norm = lax.rsqrt(jnp.maximum(deg, 1.0))
        norm_ref[...] = norm
        g0 = feat_ref[...] * norm
        for c in range(2):
            for q in range(2):
                g4_ref[c, q] = g0[:, (2 * c + q) * _DQ:(2 * c + q + 1) * _DQ]

    return pl.pallas_call(
        body,
        out_shape=(jax.ShapeDtypeStruct((_N, 1), jnp.float32),
                   jax.ShapeDtypeStruct((_NC, 2, _N, _DQ), jnp.float32)),
    )(degw, feat)


def _tc_mid(acc, norm):
    def body(acc_ref, norm_ref, g4_ref):
        t = jnp.concatenate(
            [acc_ref[0, 0], acc_ref[0, 1], acc_ref[1, 0], acc_ref[1, 1]],
            axis=1) * norm_ref[...]
        m = jnp.mean(t, axis=0, keepdims=True)
        d = t - m
        var = jnp.sum(d * d, axis=0, keepdims=True) / (_N - 1)
        g = d * lax.rsqrt(var) * norm_ref[...]
        for c in range(2):
            for q in range(2):
                g4_ref[c, q] = g[:, (2 * c + q) * _DQ:(2 * c + q + 1) * _DQ]

    return pl.pallas_call(
        body,
        out_shape=jax.ShapeDtypeStruct((_NC, 2, _N, _DQ), jnp.float32),
    )(acc, norm)


def _tc_final(acc, norm, W, b):
    def body(acc_ref, norm_ref, W_ref, b_ref, out_ref):
        t = jnp.concatenate(
            [acc_ref[0, 0], acc_ref[0, 1], acc_ref[1, 0], acc_ref[1, 1]],
            axis=1) * norm_ref[...]
        m = jnp.mean(t, axis=0, keepdims=True)
        d = t - m
        var = jnp.sum(d * d, axis=0, keepdims=True) / (_N - 1)
        hn = d * lax.rsqrt(var)
        out_ref[...] = (
            jnp.dot(hn, W_ref[...], preferred_element_type=jnp.float32)
            + b_ref[...]
        )

    return pl.pallas_call(
        body,
        out_shape=jax.ShapeDtypeStruct((_N, _D), jnp.float32),
    )(acc, norm, W, b)


def kernel(feat, edge_index, W, b):
    src = edge_index[0]
    dst3d = edge_index[1].reshape(_NW, _NCHD, _CH)
    dsth = edge_index[1].reshape(_NS, _NCH, _CH)
    ones16 = jnp.ones((_CH, 16), jnp.float32)
    zeros16 = jnp.zeros((_NPS, 16), jnp.float32)
    zerosQ = jnp.zeros((_NPS, _DQ), jnp.float32)

    degw = _sc_degree(dst3d, ones16, zeros16).reshape(_NC, _N, 16)
    norm, g0 = _tc_prep(degw, feat)

    def hop(g4):
        acc = _sc_hop(g4, src, dsth, zerosQ)
        return acc.reshape(_NC, 2, _N, _DQ)

    g1 = _tc_mid(hop(g0), norm)
    return _tc_final(hop(g1), norm, W, b.reshape(1, _D))


# speedup vs baseline: 11.7520x; 1.3916x over previous
import functools

import jax
import jax.numpy as jnp
from jax import lax
from jax.experimental import pallas as pl
from jax.experimental.pallas import tpu as pltpu
from jax.experimental.pallas import tpu_sc as plsc

_N = 10000
_E = 320000
_D = 128
_DH = _D // 2

_NC = 2
_NS = 16
_NW = _NC * _NS
_CH = 80

_EPW = _E // _NW
_NCHD = _EPW // _CH

_EPS = _E // _NS
_NCH = _EPS // _CH

_NPS = _N // _NS

_mesh = plsc.VectorSubcoreMesh(core_axis_name="c", subcore_axis_name="s")


@functools.partial(
    pl.kernel,
    out_type=jax.ShapeDtypeStruct((_NC, _NS, _NPS, 16), jnp.float32),
    mesh=_mesh,
    scratch_types=[
        pltpu.VMEM((_NCHD, _CH), jnp.int32),
        pltpu.VMEM((_CH, 16), jnp.float32),
        pltpu.VMEM_SHARED((_N, 16), jnp.float32),
        pltpu.SemaphoreType.DMA,
    ],
    compiler_params=pltpu.CompilerParams(use_tc_tiling_on_sc=False),
)
def _sc_degree(dst_hbm, ones_hbm, zeros_hbm, out_hbm, didx, ones_v, acc, dsem):
    cid = lax.axis_index("c")
    sid = lax.axis_index("s")
    wid = sid * _NC + cid
    pltpu.sync_copy(ones_hbm, ones_v)
    pltpu.sync_copy(dst_hbm.at[wid], didx)
    pltpu.sync_copy(zeros_hbm, acc.at[pl.ds(sid * _NPS, _NPS)])
    plsc.subcore_barrier()

    _W = 16

    def dstart(c):
        pltpu.async_copy(ones_v, acc.at[didx.at[c]], dsem, add=True)

    def dwait(c):
        pltpu.make_async_copy(ones_v, acc.at[didx.at[c]], dsem).wait()

    for c in range(_W):
        dstart(c)

    def body(c, carry):
        dstart(c)
        dwait(c - _W)
        return carry

    lax.fori_loop(_W, _NCHD, body, 0)
    for c in range(_NCHD - _W, _NCHD):
        dwait(c)
    plsc.subcore_barrier()
    pltpu.sync_copy(acc.at[pl.ds(sid * _NPS, _NPS)], out_hbm.at[cid, sid])


@functools.partial(
    pl.kernel,
    out_type=jax.ShapeDtypeStruct((_NC, _NS, _NPS, _DH), jnp.float32),
    mesh=_mesh,
    scratch_types=[
        pltpu.VMEM((_EPS,), jnp.int32),
        pltpu.VMEM((_NCH, _CH), jnp.int32),
        pltpu.VMEM((9, _CH, _DH), jnp.float32),
        pltpu.VMEM_SHARED((_N, _DH), jnp.float32),
        [pltpu.SemaphoreType.DMA] * 9,
        [pltpu.SemaphoreType.DMA] * 9,
    ],
    compiler_params=pltpu.CompilerParams(use_tc_tiling_on_sc=False),
)
def _sc_hop(g2_hbm, src_hbm, dst_hbm, zeros_hbm, out_hbm,
            sidx, didx, rows, acc, gsems, ssems):
    cid = lax.axis_index("c")
    sid = lax.axis_index("s")
    cp_s = pltpu.async_copy(src_hbm.at[pl.ds(sid * _EPS, _EPS)], sidx, gsems[0])
    cp_d = pltpu.async_copy(dst_hbm.at[sid], didx, gsems[1])
    cp_z = pltpu.async_copy(zeros_hbm, acc.at[pl.ds(sid * _NPS, _NPS)], gsems[2])
    cp_s.wait()

    civ = jnp.full((16,), cid, jnp.int32)

    def tbody(i, carry):
        v = sidx[pl.ds(i * 16, 16)]
        sidx[pl.ds(i * 16, 16)] = v + v + civ
        return carry

    lax.fori_loop(0, _EPS // 16, tbody, 0)
    cp_d.wait()
    cp_z.wait()
    plsc.subcore_barrier()

    _R = 9
    _G = 7

    def start_gather(c, b):
        pltpu.async_copy(g2_hbm.at[sidx.at[pl.ds(c * _CH, _CH)]],
                         rows.at[b], gsems[b])

    def wait_gather(c, b):
        pltpu.make_async_copy(g2_hbm.at[sidx.at[pl.ds(c * _CH, _CH)]],
                              rows.at[b], gsems[b]).wait()

    def start_scatter(c, b):
        pltpu.async_copy(rows.at[b], acc.at[didx.at[c]], ssems[b], add=True)

    def wait_scatter(c, b):
        pltpu.make_async_copy(rows.at[b], acc.at[didx.at[c]], ssems[b]).wait()

    for c in range(_G):
        start_gather(c, c % _R)
    for c in range(_G):
        if c >= _R - _G:
            wait_scatter(c - (_R - _G), (c - (_R - _G)) % _R)
        start_gather(c + _G, (c + _G) % _R)
        wait_gather(c, c % _R)
        start_scatter(c, c % _R)

    n_steady = (_NCH - 2 * _G) // _R * _R

    def body(i, carry):
        for j in range(_R):
            c = i * _R + j + _G
            bw = (j + 2 * _G) % _R
            wait_scatter(c - (_R - _G), bw)
            start_gather(c + _G, bw)
            b = (j + _G) % _R
            wait_gather(c, b)
            start_scatter(c, b)
        return carry

    lax.fori_loop(0, n_steady // _R, body, 0)
    for c in range(_G + n_steady, _NCH):
        wait_scatter(c - (_R - _G), (c - (_R - _G)) % _R)
        if c + _G < _NCH:
            start_gather(c + _G, (c + _G) % _R)
        wait_gather(c, c % _R)
        start_scatter(c, c % _R)
    for c in range(_NCH - (_R - _G), _NCH):
        wait_scatter(c, c % _R)
    plsc.subcore_barrier()
    pltpu.sync_copy(acc.at[pl.ds(sid * _NPS, _NPS)], out_hbm.at[cid, sid])


def _tc_prep(degw, feat):
    def body(degw_ref, feat_ref, norm_ref, g0_ref):
        deg = degw_ref[0, :, 0:1] + degw_ref[1, :, 0:1]
        norm = lax.rsqrt(jnp.maximum(deg, 1.0))
        norm_ref[...] = norm
        g0_ref[...] = feat_ref[...] * norm

    return pl.pallas_call(
        body,
        out_shape=(jax.ShapeDtypeStruct((_N, 1), jnp.float32),
                   jax.ShapeDtypeStruct((_N, _D), jnp.float32)),
    )(degw, feat)


def _tc_mid(acc, norm):
    def body(acc_ref, norm_ref, g_ref):
        t = jnp.concatenate([acc_ref[0], acc_ref[1]], axis=1) * norm_ref[...]
        m = jnp.mean(t, axis=0, keepdims=True)
        d = t - m
        var = jnp.sum(d * d, axis=0, keepdims=True) / (_N - 1)
        g_ref[...] = d * lax.rsqrt(var) * norm_ref[...]

    return pl.pallas_call(
        body,
        out_shape=jax.ShapeDtypeStruct((_N, _D), jnp.float32),
    )(acc, norm)


def _tc_final(acc, norm, W, b):
    def body(acc_ref, norm_ref, W_ref, b_ref, out_ref):
        t = jnp.concatenate([acc_ref[0], acc_ref[1]], axis=1) * norm_ref[...]
        m = jnp.mean(t, axis=0, keepdims=True)
        d = t - m
        var = jnp.sum(d * d, axis=0, keepdims=True) / (_N - 1)
        hn = d * lax.rsqrt(var)
        out_ref[...] = (
            jnp.dot(hn, W_ref[...], preferred_element_type=jnp.float32)
            + b_ref[...]
        )

    return pl.pallas_call(
        body,
        out_shape=jax.ShapeDtypeStruct((_N, _D), jnp.float32),
    )(acc, norm, W, b)


def kernel(feat, edge_index, W, b):
    src = edge_index[0]
    dst3d = edge_index[1].reshape(_NW, _NCHD, _CH)
    dsth = edge_index[1].reshape(_NS, _NCH, _CH)
    ones16 = jnp.ones((_CH, 16), jnp.float32)
    zeros16 = jnp.zeros((_NPS, 16), jnp.float32)
    zerosH = jnp.zeros((_NPS, _DH), jnp.float32)

    degw = _sc_degree(dst3d, ones16, zeros16).reshape(_NC, _N, 16)
    norm, g0 = _tc_prep(degw, feat)

    def hop(g):
        acc = _sc_hop(g.reshape(2 * _N, _DH), src, dsth, zerosH)
        return acc.reshape(_NC, _N, _DH)

    g1 = _tc_mid(hop(g0), norm)
    return _tc_final(hop(g1), norm, W, b.reshape(1, _D))
